# baseline (speedup 1.0000x reference)
import functools

import jax
import jax.numpy as jnp
from jax import lax
from jax.experimental import pallas as pl
from jax.experimental.pallas import tpu as pltpu
from jax.experimental.pallas import tpu_sc as plsc

_NC = 2
_NS = 16
_NW = _NC * _NS
_L = 16


def _pool_kernel(n_pairs, n, nodes, nodes_pad):
    pairs_per_w = n_pairs // _NW
    n_vec = nodes_pad // _L
    mesh = plsc.VectorSubcoreMesh(core_axis_name="c", subcore_axis_name="s")

    @functools.partial(
        pl.kernel,
        mesh=mesh,
        compiler_params=pltpu.CompilerParams(
            needs_layout_passes=False, use_tc_tiling_on_sc=False
        ),
        out_type=jax.ShapeDtypeStruct((n_pairs, nodes), jnp.float32),
        scratch_types=[
            pltpu.VMEM((n,), jnp.float32),
            pltpu.VMEM((7 * nodes_pad,), jnp.int32),
            pltpu.VMEM((nodes_pad,), jnp.float32),
        ],
    )
    def body(x_hbm, idx_hbm, out_hbm, xrow, idxv, outv):
        wid = lax.axis_index("s") * _NC + lax.axis_index("c")
        base = wid * pairs_per_w
        pltpu.sync_copy(idx_hbm, idxv)

        def per_pair(p, carry):
            pair = base + p
            pltpu.sync_copy(x_hbm.at[pair], xrow)

            def per_vec(j, carry2):
                off = j * _L
                acc = jnp.zeros((_L,), jnp.float32)
                for k in range(7):
                    iv = idxv[pl.ds(k * nodes_pad + off, _L)]
                    acc = acc + plsc.load_gather(xrow, [iv])
                outv[pl.ds(off, _L)] = acc * jnp.float32(1.0 / 7.0)
                return carry2

            lax.fori_loop(0, n_vec, per_vec, 0, unroll=False)
            pltpu.sync_copy(outv.at[pl.ds(0, nodes)], out_hbm.at[pair])
            return carry

        lax.fori_loop(0, pairs_per_w, per_pair, 0, unroll=False)

    return body


def kernel(x, neigh_orders):
    B, C, N = x.shape
    nodes = (N + 6) // 4
    nodes_pad = ((nodes + _L - 1) // _L) * _L
    n_pairs = B * C

    x2 = x.reshape(n_pairs, N)
    idx = neigh_orders[: nodes * 7].reshape(nodes, 7).T
    idx = jnp.pad(idx, ((0, 0), (0, nodes_pad - nodes))).reshape(-1)

    out = _pool_kernel(n_pairs, N, nodes, nodes_pad)(x2, idx)
    return out.reshape(B, C, nodes)
